# baseline (speedup 1.0000x reference)
import contextlib
import functools

import jax
import jax.numpy as jnp
import numpy as np
from jax.experimental import pallas as pl
from jax.experimental.pallas import tpu as pltpu

_MASK_PROB = 0.2
_MASK_SPAN = 10

_START_CACHE = {}


def _get_starts(B, T):
    if (B, T) not in _START_CACHE:
        num_spans = max(1, int(_MASK_PROB * (T / _MASK_SPAN)))
        max_start = max(1, T - _MASK_SPAN)
        try:
            dev_ctx = jax.default_device(jax.local_devices(backend="cpu")[0])
        except Exception:
            dev_ctx = contextlib.nullcontext()
        with jax.ensure_compile_time_eval(), dev_ctx:
            keys = jax.random.split(jax.random.key(42), B)
            rows = [np.asarray(jax.random.permutation(k, max_start))[:num_spans]
                    for k in keys]
        _START_CACHE[(B, T)] = np.stack(rows).astype(np.int32)
    return _START_CACHE[(B, T)]


def _mask_body(span, starts_ref, emb_ref, z_ref, out_ref, m_ref):
    t_blk = pl.program_id(1)
    tt = out_ref.shape[1]
    s_pad = starts_ref.shape[2]
    st = starts_ref[0]
    ti = jax.lax.broadcasted_iota(jnp.int32, (tt, s_pad), 0) + t_blk * tt
    hit = (ti >= st) & (ti < st + span)
    mrow = jnp.any(hit, axis=1, keepdims=True)
    m_ref[0, 0] = mrow.astype(jnp.int32)
    out_ref[0] = jnp.where(mrow, emb_ref[...], z_ref[0])


def kernel(z_t, mask_emb):
    B, T, D = z_t.shape
    starts = _get_starts(B, T)
    S = starts.shape[1]
    S_PAD = -(-S // 128) * 128
    starts3 = np.full((B, 1, S_PAD), T, dtype=np.int32)
    starts3[:, 0, :S] = starts
    starts3 = jnp.asarray(starts3)

    TT = 1024
    grid = (B, T // TT)
    out, mask4 = pl.pallas_call(
        functools.partial(_mask_body, _MASK_SPAN),
        grid=grid,
        in_specs=[
            pl.BlockSpec((1, 1, S_PAD), lambda b, t: (b, 0, 0)),
            pl.BlockSpec((1, D), lambda b, t: (0, 0)),
            pl.BlockSpec((1, TT, D), lambda b, t: (b, t, 0)),
        ],
        out_specs=[
            pl.BlockSpec((1, TT, D), lambda b, t: (b, t, 0)),
            pl.BlockSpec((1, 1, TT, 1), lambda b, t: (b, t, 0, 0)),
        ],
        out_shape=[
            jax.ShapeDtypeStruct((B, T, D), z_t.dtype),
            jax.ShapeDtypeStruct((B, T // TT, TT, 1), jnp.int32),
        ],
        compiler_params=pltpu.CompilerParams(
            dimension_semantics=("parallel", "parallel"),
            vmem_limit_bytes=100 * 1024 * 1024,
        ),
    )(starts3, mask_emb.reshape(1, D), z_t)
    return out, mask4.reshape(B, T).astype(bool)


# speedup vs baseline: 1.0209x; 1.0209x over previous
import contextlib
import functools

import jax
import jax.numpy as jnp
import numpy as np
from jax.experimental import pallas as pl
from jax.experimental.pallas import tpu as pltpu

_MASK_PROB = 0.2
_MASK_SPAN = 10

_START_CACHE = {}


def _get_starts(B, T):
    if (B, T) not in _START_CACHE:
        num_spans = max(1, int(_MASK_PROB * (T / _MASK_SPAN)))
        max_start = max(1, T - _MASK_SPAN)
        try:
            dev_ctx = jax.default_device(jax.local_devices(backend="cpu")[0])
        except Exception:
            dev_ctx = contextlib.nullcontext()
        with jax.ensure_compile_time_eval(), dev_ctx:
            keys = jax.random.split(jax.random.key(42), B)
            rows = [np.asarray(jax.random.permutation(k, max_start))[:num_spans]
                    for k in keys]
        _START_CACHE[(B, T)] = np.stack(rows).astype(np.int32)
    return _START_CACHE[(B, T)]


def _mask_body(span, starts_ref, emb_ref, z_ref, out_ref, m_ref):
    t_blk = pl.program_id(1)
    tt = out_ref.shape[1]
    s_pad = starts_ref.shape[2]
    st = starts_ref[0]
    ti = jax.lax.broadcasted_iota(jnp.int32, (tt, s_pad), 0) + t_blk * tt
    hit = (ti >= st) & (ti < st + span)
    mrow = jnp.any(hit, axis=1, keepdims=True)
    m_ref[0, 0] = mrow.astype(jnp.int32)
    out_ref[0] = jnp.where(mrow, emb_ref[...], z_ref[0])


def kernel(z_t, mask_emb):
    B, T, D = z_t.shape
    starts = _get_starts(B, T)
    S = starts.shape[1]
    S_PAD = -(-S // 128) * 128
    starts3 = np.full((B, 1, S_PAD), T, dtype=np.int32)
    starts3[:, 0, :S] = starts
    starts3 = jnp.asarray(starts3)

    TT = 2048
    grid = (B, T // TT)
    out, mask4 = pl.pallas_call(
        functools.partial(_mask_body, _MASK_SPAN),
        grid=grid,
        in_specs=[
            pl.BlockSpec((1, 1, S_PAD), lambda b, t: (b, 0, 0)),
            pl.BlockSpec((1, D), lambda b, t: (0, 0)),
            pl.BlockSpec((1, TT, D), lambda b, t: (b, t, 0)),
        ],
        out_specs=[
            pl.BlockSpec((1, TT, D), lambda b, t: (b, t, 0)),
            pl.BlockSpec((1, 1, TT, 1), lambda b, t: (b, t, 0, 0)),
        ],
        out_shape=[
            jax.ShapeDtypeStruct((B, T, D), z_t.dtype),
            jax.ShapeDtypeStruct((B, T // TT, TT, 1), jnp.int32),
        ],
        compiler_params=pltpu.CompilerParams(
            dimension_semantics=("parallel", "parallel"),
            vmem_limit_bytes=100 * 1024 * 1024,
        ),
    )(starts3, mask_emb.reshape(1, D), z_t)
    return out, mask4.reshape(B, T).astype(bool)


# speedup vs baseline: 1.1702x; 1.1463x over previous
import contextlib
import functools

import jax
import jax.numpy as jnp
import numpy as np
from jax.experimental import pallas as pl
from jax.experimental.pallas import tpu as pltpu

_MASK_PROB = 0.2
_MASK_SPAN = 10

_START_CACHE = {}


def _get_starts(B, T):
    if (B, T) not in _START_CACHE:
        num_spans = max(1, int(_MASK_PROB * (T / _MASK_SPAN)))
        max_start = max(1, T - _MASK_SPAN)
        try:
            dev_ctx = jax.default_device(jax.local_devices(backend="cpu")[0])
        except Exception:
            dev_ctx = contextlib.nullcontext()
        with jax.ensure_compile_time_eval(), dev_ctx:
            keys = jax.random.split(jax.random.key(42), B)
            rows = [np.asarray(jax.random.permutation(k, max_start))[:num_spans]
                    for k in keys]
        _START_CACHE[(B, T)] = np.stack(rows).astype(np.int32)
    return _START_CACHE[(B, T)]


def _mask_body(span, starts_ref, starts_col_ref, emb_ref, z_ref, out_ref, m_ref):
    t_blk = pl.program_id(1)
    tt = out_ref.shape[1]
    s_pad = starts_ref.shape[2]
    st = starts_ref[0]
    ti = jax.lax.broadcasted_iota(jnp.int32, (tt, s_pad), 0) + t_blk * tt
    hit = (ti >= st) & (ti < st + span)
    mrow = jnp.any(hit, axis=1, keepdims=True)
    s_padc = starts_col_ref.shape[1]
    stc = starts_col_ref[0]
    tic = jax.lax.broadcasted_iota(jnp.int32, (s_padc, tt), 1) + t_blk * tt
    hitc = (tic >= stc) & (tic < stc + span)
    m_ref[0] = jnp.any(hitc, axis=0, keepdims=True)
    out_ref[0] = jnp.where(mrow, emb_ref[...], z_ref[0])


def kernel(z_t, mask_emb):
    B, T, D = z_t.shape
    starts = _get_starts(B, T)
    S = starts.shape[1]
    S_PAD = -(-S // 128) * 128
    starts3 = np.full((B, 1, S_PAD), T, dtype=np.int32)
    starts3[:, 0, :S] = starts
    starts3 = jnp.asarray(starts3)
    S_PADC = -(-S // 8) * 8
    starts_col = np.full((B, S_PADC, 1), T, dtype=np.int32)
    starts_col[:, :S, 0] = starts
    starts_col = jnp.asarray(starts_col)

    TT = 2048
    grid = (B, T // TT)
    out, mask = pl.pallas_call(
        functools.partial(_mask_body, _MASK_SPAN),
        grid=grid,
        in_specs=[
            pl.BlockSpec((1, 1, S_PAD), lambda b, t: (b, 0, 0)),
            pl.BlockSpec((1, S_PADC, 1), lambda b, t: (b, 0, 0)),
            pl.BlockSpec((1, D), lambda b, t: (0, 0)),
            pl.BlockSpec((1, TT, D), lambda b, t: (b, t, 0)),
        ],
        out_specs=[
            pl.BlockSpec((1, TT, D), lambda b, t: (b, t, 0)),
            pl.BlockSpec((1, 1, TT), lambda b, t: (b, 0, t)),
        ],
        out_shape=[
            jax.ShapeDtypeStruct((B, T, D), z_t.dtype),
            jax.ShapeDtypeStruct((B, 1, T), jnp.bool_),
        ],
        compiler_params=pltpu.CompilerParams(
            dimension_semantics=("parallel", "parallel"),
            vmem_limit_bytes=100 * 1024 * 1024,
        ),
    )(starts3, starts_col, mask_emb.reshape(1, D), z_t)
    return out, mask.reshape(B, T)
